# baseline (speedup 1.0000x reference)
import functools
import math

import jax
import jax.numpy as jnp
from jax import lax
from jax.experimental import pallas as pl
from jax.experimental.pallas import tpu as pltpu
from jax.experimental.pallas import tpu_sc as plsc

NUM_EMBEDDINGS = 1000000
EMBED_DIM = 64
PAD_DIM = 128
TOTAL_ROWS = 4096 * 200
NUM_WORKERS = 32
ROWS_PER_WORKER = TOTAL_ROWS // NUM_WORKERS
TILE_ROWS = 64
NUM_TILES = ROWS_PER_WORKER // TILE_ROWS
NBUF = 5
FIRE_AHEAD = 3
SCALE = math.sqrt(EMBED_DIM)
LANES = 16
VECS_PER_ROW = EMBED_DIM // LANES
UNROLL = 16


def _perm(v, idx):
    return lax.gather(
        v,
        idx[:, None],
        dimension_numbers=lax.GatherDimensionNumbers(
            offset_dims=(), collapsed_slice_dims=(0,), start_index_map=(0,)),
        slice_sizes=(1,),
        mode=lax.GatherScatterMode.PROMISE_IN_BOUNDS,
    )


def _rsqrt_newton(s):
    i = lax.bitcast_convert_type(s, jnp.int32)
    i = jnp.int32(0x5F3759DF) - lax.shift_right_arithmetic(i, 1)
    y = lax.bitcast_convert_type(i, jnp.float32)
    half = jnp.float32(0.5) * s
    for _ in range(2):
        y = y * (jnp.float32(1.5) - half * y * y)
    return y


def _make_kernel():
    mesh = plsc.VectorSubcoreMesh(core_axis_name="c", subcore_axis_name="s")

    @functools.partial(
        pl.kernel,
        mesh=mesh,
        out_type=jax.ShapeDtypeStruct((TOTAL_ROWS // TILE_ROWS,
                                       TILE_ROWS, EMBED_DIM), jnp.float32),
        scratch_types=[pltpu.VMEM((TILE_ROWS,), jnp.int32)
                       for _ in range(NBUF)]
        + [
            pltpu.VMEM((NBUF, 1, TILE_ROWS, PAD_DIM), jnp.float32),
            pltpu.VMEM((NBUF, 1, TILE_ROWS, EMBED_DIM), jnp.float32),
        ]
        + [pltpu.SemaphoreType.DMA] * (2 * NBUF),
    )
    def emb_kernel(idx_hbm, table_hbm, out_hbm, *refs):
        idx_refs = refs[:NBUF]
        rows_v = refs[NBUF]
        cmp_v = refs[NBUF + 1]
        sems = refs[NBUF + 2:]
        gsems = sems[:NBUF]
        osems = sems[NBUF:]
        wid = lax.axis_index("s") * 2 + lax.axis_index("c")
        t0_row = wid * NUM_TILES

        def idx_copy(t, p):
            pltpu.sync_copy(idx_hbm.at[t0_row + t, 0], idx_refs[p])

        def fire_gather(p):
            pltpu.async_copy(
                table_hbm.at[idx_refs[p]],
                rows_v.at[p, 0],
                gsems[p],
            )

        def wait_gather(p):
            pltpu.make_async_copy(
                table_hbm.at[idx_refs[p]],
                rows_v.at[p, 0],
                gsems[p],
            ).wait()

        def fire_out(t, p):
            pltpu.async_copy(
                cmp_v.at[p, 0],
                out_hbm.at[t0_row + t],
                osems[p],
            )

        def wait_out(p):
            pltpu.make_async_copy(
                cmp_v.at[p, 0],
                out_hbm.at[t0_row],
                osems[p],
            ).wait()

        def compute(p):
            lanes = lax.iota(jnp.int32, LANES)

            def row_pack(r0):
                vs = [[rows_v[p, 0, r0 + q, pl.ds(k * LANES, LANES)]
                       for k in range(VECS_PER_ROW)] for q in range(4)]
                f = []
                for q in range(4):
                    sq = vs[q][0] * vs[q][0]
                    for k in range(1, VECS_PER_ROW):
                        sq = sq + vs[q][k] * vs[q][k]
                    g = sq + _perm(sq, lanes ^ 8)
                    g = g + _perm(g, lanes ^ 4)
                    f.append(g)
                m = jnp.where(lanes < 4, f[0],
                              jnp.where(lanes < 8, f[1],
                                        jnp.where(lanes < 12, f[2], f[3])))
                m = m + _perm(m, lanes ^ 2)
                m = m + _perm(m, lanes ^ 1)
                m = jnp.maximum(m, jnp.float32(1e-24))
                y = _rsqrt_newton(m) * jnp.float32(SCALE)
                for q in range(4):
                    fac = _perm(y, jnp.full((LANES,), 4 * q, jnp.int32))
                    for k in range(VECS_PER_ROW):
                        cmp_v[p, 0, r0 + q, pl.ds(k * LANES, LANES)] = (
                            vs[q][k] * fac)

            def row_body(i, _):
                for u in range(UNROLL // 4):
                    row_pack(i * UNROLL + u * 4)
                return _

            lax.fori_loop(0, TILE_ROWS // UNROLL, row_body, None)

        for t0 in range(FIRE_AHEAD):
            idx_copy(jnp.int32(t0), t0)
            fire_gather(t0)

        def ring_body(i, _):
            for par in range(NBUF):
                t = i * NBUF + par
                wait_gather(par)
                compute(par)
                fire_out(t, par)
                q = (par + FIRE_AHEAD) % NBUF

                @pl.when(t >= NBUF - FIRE_AHEAD)
                def _():
                    wait_out(q)

                @pl.when(t + FIRE_AHEAD < NUM_TILES)
                def _():
                    idx_copy(t + FIRE_AHEAD, q)
                    fire_gather(q)
            return _

        lax.fori_loop(0, NUM_TILES // NBUF, ring_body, None)

        wait_out((NUM_TILES - 2) % NBUF)
        wait_out((NUM_TILES - 1) % NBUF)

    return emb_kernel


_EMB_KERNEL = _make_kernel()


@jax.jit
def kernel(x, raw_embedding):
    table128 = jnp.pad(raw_embedding, ((0, 0), (0, PAD_DIM - EMBED_DIM)))
    idx3d = x.reshape(TOTAL_ROWS // TILE_ROWS, 1,
                      TILE_ROWS).astype(jnp.int32)
    out = _EMB_KERNEL(idx3d, table128)
    return out.reshape(x.shape[0], x.shape[1], EMBED_DIM)


# speedup vs baseline: 1.2292x; 1.2292x over previous
import functools
import math

import jax
import jax.numpy as jnp
from jax import lax
from jax.experimental import pallas as pl
from jax.experimental.pallas import tpu as pltpu
from jax.experimental.pallas import tpu_sc as plsc

NUM_EMBEDDINGS = 1000000
EMBED_DIM = 64
PAD_DIM = 128
TOTAL_ROWS = 4096 * 200
NUM_WORKERS = 32
ROWS_PER_WORKER = TOTAL_ROWS // NUM_WORKERS
TILE_ROWS = 80
NUM_TILES = ROWS_PER_WORKER // TILE_ROWS
NBUF = 5
FIRE_AHEAD = 3
SCALE = math.sqrt(EMBED_DIM)
LANES = 16
VECS_PER_ROW = EMBED_DIM // LANES
UNROLL = 8


def _perm(v, idx):
    return lax.gather(
        v,
        idx[:, None],
        dimension_numbers=lax.GatherDimensionNumbers(
            offset_dims=(), collapsed_slice_dims=(0,), start_index_map=(0,)),
        slice_sizes=(1,),
        mode=lax.GatherScatterMode.PROMISE_IN_BOUNDS,
    )


def _rsqrt_newton(s):
    i = lax.bitcast_convert_type(s, jnp.int32)
    i = jnp.int32(0x5F3759DF) - lax.shift_right_arithmetic(i, 1)
    y = lax.bitcast_convert_type(i, jnp.float32)
    half = jnp.float32(0.5) * s
    for _ in range(1):
        y = y * (jnp.float32(1.5) - half * y * y)
    return y


def _make_kernel():
    mesh = plsc.VectorSubcoreMesh(core_axis_name="c", subcore_axis_name="s")

    @functools.partial(
        pl.kernel,
        mesh=mesh,
        out_type=jax.ShapeDtypeStruct((TOTAL_ROWS // TILE_ROWS,
                                       TILE_ROWS, EMBED_DIM), jnp.float32),
        scratch_types=[pltpu.VMEM((TILE_ROWS,), jnp.int32)
                       for _ in range(NBUF)]
        + [
            pltpu.VMEM((NBUF, 1, TILE_ROWS, PAD_DIM), jnp.float32),
            pltpu.VMEM((NBUF, 1, TILE_ROWS, EMBED_DIM), jnp.float32),
        ]
        + [pltpu.SemaphoreType.DMA] * (2 * NBUF),
    )
    def emb_kernel(idx_hbm, table_hbm, out_hbm, *refs):
        idx_refs = refs[:NBUF]
        rows_v = refs[NBUF]
        cmp_v = refs[NBUF + 1]
        sems = refs[NBUF + 2:]
        gsems = sems[:NBUF]
        osems = sems[NBUF:]
        wid = lax.axis_index("s") * 2 + lax.axis_index("c")
        t0_row = wid * NUM_TILES

        def idx_copy(t, p):
            pltpu.sync_copy(idx_hbm.at[t0_row + t, 0], idx_refs[p])

        def fire_gather(p):
            pltpu.async_copy(
                table_hbm.at[idx_refs[p]],
                rows_v.at[p, 0],
                gsems[p],
            )

        def wait_gather(p):
            pltpu.make_async_copy(
                table_hbm.at[idx_refs[p]],
                rows_v.at[p, 0],
                gsems[p],
            ).wait()

        def fire_out(t, p):
            pltpu.async_copy(
                cmp_v.at[p, 0],
                out_hbm.at[t0_row + t],
                osems[p],
            )

        def wait_out(p):
            pltpu.make_async_copy(
                cmp_v.at[p, 0],
                out_hbm.at[t0_row],
                osems[p],
            ).wait()

        def compute(p):
            lanes = lax.iota(jnp.int32, LANES)

            def row_pack(r0):
                vs = [[rows_v[p, 0, r0 + q, pl.ds(k * LANES, LANES)]
                       for k in range(VECS_PER_ROW)] for q in range(4)]
                f = []
                for q in range(4):
                    sq = vs[q][0] * vs[q][0]
                    for k in range(1, VECS_PER_ROW):
                        sq = sq + vs[q][k] * vs[q][k]
                    g = sq + _perm(sq, lanes ^ 8)
                    g = g + _perm(g, lanes ^ 4)
                    f.append(g)
                m = jnp.where(lanes < 4, f[0],
                              jnp.where(lanes < 8, f[1],
                                        jnp.where(lanes < 12, f[2], f[3])))
                m = m + _perm(m, lanes ^ 2)
                m = m + _perm(m, lanes ^ 1)
                m = jnp.maximum(m, jnp.float32(1e-24))
                y = _rsqrt_newton(m) * jnp.float32(SCALE)
                for q in range(4):
                    fac = _perm(y, jnp.full((LANES,), 4 * q, jnp.int32))
                    for k in range(VECS_PER_ROW):
                        cmp_v[p, 0, r0 + q, pl.ds(k * LANES, LANES)] = (
                            vs[q][k] * fac)

            def row_body(i, _):
                for u in range(UNROLL // 4):
                    row_pack(i * UNROLL + u * 4)
                return _

            lax.fori_loop(0, TILE_ROWS // UNROLL, row_body, None)

        for t0 in range(FIRE_AHEAD):
            idx_copy(jnp.int32(t0), t0)
            fire_gather(t0)

        def ring_body(i, _):
            for par in range(NBUF):
                t = i * NBUF + par
                wait_gather(par)
                compute(par)
                fire_out(t, par)
                q = (par + FIRE_AHEAD) % NBUF

                @pl.when(t >= NBUF - FIRE_AHEAD)
                def _():
                    wait_out(q)

                @pl.when(t + FIRE_AHEAD < NUM_TILES)
                def _():
                    idx_copy(t + FIRE_AHEAD, q)
                    fire_gather(q)
            return _

        lax.fori_loop(0, NUM_TILES // NBUF, ring_body, None)

        wait_out((NUM_TILES - 2) % NBUF)
        wait_out((NUM_TILES - 1) % NBUF)

    return emb_kernel


_EMB_KERNEL = _make_kernel()


@jax.jit
def kernel(x, raw_embedding):
    table128 = jnp.pad(raw_embedding, ((0, 0), (0, PAD_DIM - EMBED_DIM)))
    idx3d = x.reshape(TOTAL_ROWS // TILE_ROWS, 1,
                      TILE_ROWS).astype(jnp.int32)
    out = _EMB_KERNEL(idx3d, table128)
    return out.reshape(x.shape[0], x.shape[1], EMBED_DIM)
